# Initial kernel scaffold; baseline (speedup 1.0000x reference)
import jax
import jax.numpy as jnp
from jax.experimental import pallas as pl


def kernel(x, W1f, W1d, W2f, W2d, W3f, W3d, W4f, W4d, W5f, W5d, W6f, W6d, Wt):
    raise NotImplementedError("write your pallas kernel here")


# speedup vs baseline: 1.9208x; 1.9208x over previous
import jax
import jax.numpy as jnp
from jax.experimental import pallas as pl

EPS = 1e-6
BN_EPS = 1e-5
NEG_SLOPE = 0.2
KNN_K = 20
NEG_INF = -1e30


def _graph_feature_body(pw_ref, xt_ref, out_ref):
    xt = xt_ref[0]
    n, d = xt.shape
    col = jax.lax.broadcasted_iota(jnp.int32, (n, n), 1)
    score0 = pw_ref[0]

    def body(k, score):
        amax = jnp.max(score, axis=1, keepdims=True)
        eq = score == amax
        idx = jnp.min(jnp.where(eq, col, n), axis=1, keepdims=True)
        onehot = col == idx
        g = jax.lax.dot_general(
            onehot.astype(jnp.float32), xt, (((1,), (0,)), ((), ())),
            preferred_element_type=jnp.float32,
            precision=jax.lax.Precision.HIGHEST,
        )
        val = jnp.concatenate([g - xt, xt], axis=1)
        out_ref[0, pl.dslice(k, 1)] = val[None]
        return jnp.where(onehot, NEG_INF, score)

    jax.lax.fori_loop(0, KNN_K, body, score0)


def _graph_features(pw, xt):
    b, n, d = xt.shape
    return pl.pallas_call(
        _graph_feature_body,
        grid=(b,),
        in_specs=[
            pl.BlockSpec((1, n, n), lambda i: (i, 0, 0)),
            pl.BlockSpec((1, n, d), lambda i: (i, 0, 0)),
        ],
        out_specs=pl.BlockSpec((1, KNN_K, n, 2 * d), lambda i: (i, 0, 0, 0)),
        out_shape=jax.ShapeDtypeStruct((b, KNN_K, n, 2 * d), jnp.float32),
    )(pw, xt)


def _graph_feature(xf):
    b, d, n = xf.shape
    inner = -2.0 * jnp.einsum('bcn,bcm->bnm', xf, xf)
    xx = jnp.sum(xf ** 2, axis=1, keepdims=True)
    pw = -xx - inner - jnp.transpose(xx, (0, 2, 1))
    out = _graph_features(pw, jnp.transpose(xf, (0, 2, 1)))
    feat = out.reshape(b, KNN_K, n, 2 * (d // 3), 3)
    return jnp.transpose(feat, (0, 3, 4, 2, 1))


def _vn_linear(x, w):
    res = jnp.tensordot(w, x, axes=((1,), (1,)))
    return jnp.moveaxis(res, 0, 1)


def _vn_batchnorm(x):
    norm = jnp.linalg.norm(x, axis=2) + EPS
    axes = (0,) + tuple(range(2, norm.ndim))
    mean = jnp.mean(norm, axis=axes, keepdims=True)
    var = jnp.var(norm, axis=axes, keepdims=True)
    norm_bn = (norm - mean) / jnp.sqrt(var + BN_EPS)
    return x / norm[:, :, None] * norm_bn[:, :, None]


def _vn_lrelu(x, wf, wd):
    p = _vn_linear(x, wf)
    p = _vn_batchnorm(p)
    d = _vn_linear(x, wd)
    dot = jnp.sum(p * d, axis=2, keepdims=True)
    mask = (dot >= 0).astype(x.dtype)
    d2 = jnp.sum(d * d, axis=2, keepdims=True)
    return NEG_SLOPE * p + (1.0 - NEG_SLOPE) * (
        mask * p + (1.0 - mask) * (p - (dot / (d2 + EPS)) * d)
    )


def kernel(x, W1f, W1d, W2f, W2d, W3f, W3d, W4f, W4d, W5f, W5d, W6f, W6d, Wt):
    b, _, n = x.shape
    h = _graph_feature(x)
    h = _vn_lrelu(h, W1f, W1d)
    h = _vn_lrelu(h, W2f, W2d)
    x1 = jnp.mean(h, axis=-1)
    h = _graph_feature(x1.reshape(b, -1, n))
    h = _vn_lrelu(h, W3f, W3d)
    h = _vn_lrelu(h, W4f, W4d)
    x2 = jnp.mean(h, axis=-1)
    h = _graph_feature(x2.reshape(b, -1, n))
    h = _vn_lrelu(h, W5f, W5d)
    x3 = jnp.mean(h, axis=-1)
    x123 = jnp.concatenate([x1, x2, x3], axis=1)
    h = _vn_lrelu(x123, W6f, W6d)
    h_mean = jnp.broadcast_to(jnp.mean(h, axis=-1, keepdims=True), h.shape)
    h = jnp.mean(jnp.concatenate([h, h_mean], axis=1), axis=-1)
    h = _vn_linear(h, Wt)
    return jnp.swapaxes(h, -1, -2)
